# Initial kernel scaffold; baseline (speedup 1.0000x reference)
import jax
import jax.numpy as jnp
from jax.experimental import pallas as pl


def kernel(nodes, feat_data, neigh_idx, W):
    raise NotImplementedError("write your pallas kernel here")


# speedup vs baseline: 2.6160x; 2.6160x over previous
import functools

import jax
import jax.numpy as jnp
from jax import lax
from jax.experimental import pallas as pl
from jax.experimental.pallas import tpu as pltpu
from jax.experimental.pallas import tpu_sc as plsc

N = 50000
D = 128
S = 10
EMB = 128
B = 8192

_INFO = plsc.get_sparse_core_info()
_NC = _INFO.num_cores
_NS = _INFO.num_subcores
_NW = _NC * _NS
_B_PER_W = B // _NW
_CHUNK = 128
_NCHUNK = _B_PER_W // _CHUNK


def _sc_gather_kernel(feat_hbm, nodes_hbm, neight_hbm, self_out, nsum_out,
                      nodes_v, addr_v, nb_v, self_v, nsum_v,
                      sem_self, sem_nb, sem_acc):
  wid = lax.axis_index("s") * _NC + lax.axis_index("c")
  pltpu.sync_copy(nodes_hbm.at[pl.ds(wid * _NCHUNK, _NCHUNK)], nodes_v)
  for c in range(_NCHUNK):
    idx = nodes_v.at[c]
    cp_self = pltpu.async_copy(feat_hbm.at[idx], self_v, sem_self)
    for s in range(S):
      for g in range(_CHUNK // 16):
        addr_v[s, pl.ds(g * 16, 16)] = (
            nodes_v[c, pl.ds(g * 16, 16)] + jnp.int32(s * N))
    cps = [pltpu.async_copy(neight_hbm.at[addr_v.at[s]], nb_v.at[s], sem_nb)
           for s in range(S)]
    for cp in cps:
      cp.wait()
    pltpu.async_copy(feat_hbm.at[nb_v.at[0]], nsum_v, sem_acc).wait()
    for s in range(1, S):
      pltpu.async_copy(feat_hbm.at[nb_v.at[s]], nsum_v, sem_acc,
                       add=True).wait()
    cp_self.wait()
    base = (wid * _B_PER_W) + c * _CHUNK
    pltpu.sync_copy(self_v, self_out.at[pl.ds(base, _CHUNK)])
    pltpu.sync_copy(nsum_v, nsum_out.at[pl.ds(base, _CHUNK)])


def _tc_matmul_kernel(x_ref, n_ref, ws_ref, wn_ref, o_ref):
  acc = jnp.dot(x_ref[...], ws_ref[...], preferred_element_type=jnp.float32,
                precision=lax.Precision.HIGHEST)
  acc += jnp.dot(n_ref[...] * jnp.float32(1.0 / S), wn_ref[...],
                 preferred_element_type=jnp.float32,
                 precision=lax.Precision.HIGHEST)
  o_ref[...] = jnp.maximum(acc, 0.0)


@jax.jit
def kernel(nodes, feat_data, neigh_idx, W):
  nodes2 = nodes.astype(jnp.int32).reshape(B // 128, 128)
  neigh_t = neigh_idx.astype(jnp.int32).T.reshape(S * N)

  mesh = plsc.VectorSubcoreMesh(core_axis_name="c", subcore_axis_name="s")
  sc_gather = pl.kernel(
      _sc_gather_kernel,
      out_type=(jax.ShapeDtypeStruct((B, D), jnp.float32),
                jax.ShapeDtypeStruct((B, D), jnp.float32)),
      mesh=mesh,
      scratch_types=[
          pltpu.VMEM((_NCHUNK, _CHUNK), jnp.int32),
          pltpu.VMEM((S, _CHUNK), jnp.int32),
          pltpu.VMEM((S, _CHUNK), jnp.int32),
          pltpu.VMEM((_CHUNK, D), jnp.float32),
          pltpu.VMEM((_CHUNK, D), jnp.float32),
          pltpu.SemaphoreType.DMA,
          pltpu.SemaphoreType.DMA,
          pltpu.SemaphoreType.DMA,
      ],
  )
  self_feats, nsum = sc_gather(feat_data, nodes2, neigh_t)

  ws = W[:, :D].T
  wn = W[:, D:].T
  bm = 512
  out = pl.pallas_call(
      _tc_matmul_kernel,
      grid=(B // bm,),
      in_specs=[
          pl.BlockSpec((bm, D), lambda i: (i, 0)),
          pl.BlockSpec((bm, D), lambda i: (i, 0)),
          pl.BlockSpec((D, EMB), lambda i: (0, 0)),
          pl.BlockSpec((D, EMB), lambda i: (0, 0)),
      ],
      out_specs=pl.BlockSpec((bm, EMB), lambda i: (i, 0)),
      out_shape=jax.ShapeDtypeStruct((B, EMB), jnp.float32),
  )(self_feats, nsum, ws, wn)
  return out
